# baseline (speedup 1.0000x reference)
import functools

import jax
import jax.numpy as jnp
from jax import lax
from jax.experimental import pallas as pl
from jax.experimental.pallas import tpu as pltpu
from jax.experimental.pallas import tpu_sc as plsc

N = 2_000_000
D = 5
P = 5120
HP = P // 2
NW = 32
CH = 1600
GROUPS = CH // 16
NSLICE = 2
NS = N // NSLICE


def _make_sc_body(nch):
    def _sc_body(mse_hbm, pid_hbm, mse_out, cnt_out,
                 acc, cnt, m0, m1, p0, p1, sem):
        wid = lax.axis_index("s") * 2 + lax.axis_index("c")

        iota = lax.iota(jnp.int32, 16)
        iota16 = iota * 16
        zero_v = jnp.zeros((16,), jnp.float32)
        zero_i = jnp.zeros((16,), jnp.int32)

        def zb_acc(s, carry):
            for u in range(8):
                acc[pl.ds((s * 8 + u) * 16, 16)] = zero_v
            return carry

        def zb_cnt(s, carry):
            for u in range(8):
                cnt[pl.ds((s * 8 + u) * 16, 16)] = zero_i
            return carry

        lax.fori_loop(0, (16 * P) // 128, zb_acc, 0)
        lax.fori_loop(0, (16 * HP) // 128, zb_cnt, 0)

        def issue(c, mb, pb):
            pltpu.async_copy(mse_hbm.at[pl.ds(c * CH, CH)], mb, sem)
            pltpu.async_copy(pid_hbm.at[pl.ds(c * CH, CH)], pb, sem)

        def drain(c, mb, pb):
            pltpu.make_async_copy(
                mse_hbm.at[pl.ds(c * CH, CH)], mb, sem).wait()
            pltpu.make_async_copy(
                pid_hbm.at[pl.ds(c * CH, CH)], pb, sem).wait()

        def process(mb, pb):
            def gb(g, carry):
                for u in range(10):
                    b16 = (g * 10 + u) * 16
                    mse_v = mb[pl.ds(b16, 16)]
                    pid_v = pb[pl.ds(b16, 16)]
                    plsc.addupdate_scatter(acc, [pid_v * 16 + iota], mse_v)
                    hi = pid_v >= HP
                    col = pid_v - jnp.where(hi, HP, 0)
                    val = jnp.where(hi, 65536, 1)
                    plsc.addupdate_scatter(cnt, [col * 16 + iota], val)
                return carry
            lax.fori_loop(0, GROUPS // 10, gb, 0)

        issue(wid, m0, p0)

        def pair(j, carry):
            c0 = wid + (2 * j) * NW
            c1 = c0 + NW
            c2 = c1 + NW
            @pl.when(c0 < nch)
            def _():
                drain(c0, m0, p0)
                @pl.when(c1 < nch)
                def _():
                    issue(c1, m1, p1)
                process(m0, p0)
                @pl.when(c1 < nch)
                def _():
                    drain(c1, m1, p1)
                    @pl.when(c2 < nch)
                    def _():
                        issue(c2, m0, p0)
                    process(m1, p1)
            return carry

        lax.fori_loop(0, (nch + 2 * NW - 1) // (2 * NW), pair, 0)

        def red_acc(b, carry):
            base = b * 256
            v = plsc.load_gather(acc, [iota16 + base])
            for r in range(1, 16):
                v = v + plsc.load_gather(acc, [iota16 + (base + r)])
            acc[pl.ds(b * 16, 16)] = v
            return carry

        def red_cnt(b, carry):
            base = b * 256
            v = plsc.load_gather(cnt, [iota16 + base])
            for r in range(1, 16):
                v = v + plsc.load_gather(cnt, [iota16 + (base + r)])
            cnt[pl.ds(b * 16, 16)] = v
            return carry

        lax.fori_loop(0, P // 16, red_acc, 0)
        lax.fori_loop(0, HP // 16, red_cnt, 0)
        pltpu.sync_copy(acc.at[pl.ds(0, P)], mse_out.at[wid])
        pltpu.sync_copy(cnt.at[pl.ds(0, HP)], cnt_out.at[wid])

    return _sc_body


_sc_segment = functools.partial(
    pl.kernel,
    out_type=(jax.ShapeDtypeStruct((NW, P), jnp.float32),
              jax.ShapeDtypeStruct((NW, HP), jnp.int32)),
    mesh=plsc.VectorSubcoreMesh(core_axis_name="c", subcore_axis_name="s"),
    scratch_types=[
        pltpu.VMEM((16 * P,), jnp.float32),
        pltpu.VMEM((16 * HP,), jnp.int32),
        pltpu.VMEM((CH,), jnp.float32),
        pltpu.VMEM((CH,), jnp.float32),
        pltpu.VMEM((CH,), jnp.int32),
        pltpu.VMEM((CH,), jnp.int32),
        pltpu.SemaphoreType.DMA,
    ],
    compiler_params=pltpu.CompilerParams(needs_layout_passes=False,
                                         use_tc_tiling_on_sc=False),
)(_make_sc_body(NS // CH))


def _final_body(mse_a, mse_b, cnt_a, cnt_b, out_ref):
    sum_mse = (jnp.sum(mse_a[...], axis=0, keepdims=True)
               + jnp.sum(mse_b[...], axis=0, keepdims=True))
    low = jnp.zeros((1, HP), jnp.float32)
    high = jnp.zeros((1, HP), jnp.float32)
    for ref in (cnt_a, cnt_b):
        packed = ref[...]
        low = low + jnp.sum((packed & 0xFFFF).astype(jnp.float32),
                            axis=0, keepdims=True)
        high = high + jnp.sum(
            (lax.shift_right_logical(packed, 16) & 0xFFFF)
            .astype(jnp.float32), axis=0, keepdims=True)
    counts = jnp.concatenate([low, high], axis=1)
    pids = lax.broadcasted_iota(jnp.int32, (1, P), 1).astype(jnp.float32)
    present = (counts > 0.0) & (pids != 0.0)
    xi_sum = pids * counts
    weighted = pids * sum_mse
    terms = jnp.where(present,
                      weighted / jnp.where(xi_sum > 0.0, xi_sum, 1.0),
                      0.0)
    k_cnt = jnp.sum(present.astype(jnp.float32))
    out_ref[0, 0] = 100.0 * jnp.sum(terms) / k_cnt


def kernel(W, beta, H, pred, Y, particle_id, track_params, reconstructable):
    parts = []
    eps = jnp.float32(0.0)
    zero_i = jnp.int32(0)
    for s in range(NSLICE):
        lo, hi = s * NS, (s + 1) * NS
        mse = jnp.sum((pred[lo:hi] - track_params[lo:hi]) ** 2, axis=1) + eps
        pid_eff = jnp.where(reconstructable[lo:hi] > zero_i,
                            particle_id[lo:hi], 0)
        eps = lax.optimization_barrier(jnp.minimum(jnp.min(mse), 0.0))
        zero_i = lax.optimization_barrier(jnp.minimum(jnp.min(pid_eff), 0))
        parts.append(_sc_segment(mse, pid_eff))
    (mse_a, cnt_a), (mse_b, cnt_b) = parts
    out = pl.pallas_call(
        _final_body,
        out_shape=jax.ShapeDtypeStruct((1, 1), jnp.float32),
        out_specs=pl.BlockSpec(memory_space=pltpu.SMEM),
    )(mse_a, mse_b, cnt_a, cnt_b)
    return out[0, 0]


# speedup vs baseline: 1.0218x; 1.0218x over previous
import functools

import jax
import jax.numpy as jnp
from jax import lax
from jax.experimental import pallas as pl
from jax.experimental.pallas import tpu as pltpu
from jax.experimental.pallas import tpu_sc as plsc

N = 2_000_000
D = 5
P = 5120
HP = P // 2
NW = 32
CH = 1600
GROUPS = CH // 16
NSLICE = 2
NS = N // NSLICE


def _make_sc_body(nch):
    def _sc_body(mse_hbm, pid_hbm, mse_out, cnt_out,
                 acc, cnt, m0, m1, p0, p1, sem):
        wid = lax.axis_index("s") * 2 + lax.axis_index("c")

        iota = lax.iota(jnp.int32, 16)
        iota16 = iota * 16
        zero_v = jnp.zeros((16,), jnp.float32)
        zero_i = jnp.zeros((16,), jnp.int32)

        def zb_acc(s, carry):
            for u in range(8):
                acc[pl.ds((s * 8 + u) * 16, 16)] = zero_v
            return carry

        def zb_cnt(s, carry):
            for u in range(8):
                cnt[pl.ds((s * 8 + u) * 16, 16)] = zero_i
            return carry

        lax.fori_loop(0, (16 * P) // 128, zb_acc, 0)
        lax.fori_loop(0, (16 * HP) // 128, zb_cnt, 0)

        def issue(c, mb, pb):
            pltpu.async_copy(mse_hbm.at[pl.ds(c * CH, CH)], mb, sem)
            pltpu.async_copy(pid_hbm.at[pl.ds(c * CH, CH)], pb, sem)

        def drain(c, mb, pb):
            pltpu.make_async_copy(
                mse_hbm.at[pl.ds(c * CH, CH)], mb, sem).wait()
            pltpu.make_async_copy(
                pid_hbm.at[pl.ds(c * CH, CH)], pb, sem).wait()

        def process(mb, pb):
            def gb(g, carry):
                for u in range(10):
                    b16 = (g * 10 + u) * 16
                    mse_v = mb[pl.ds(b16, 16)]
                    pid_v = pb[pl.ds(b16, 16)]
                    plsc.addupdate_scatter(acc, [pid_v * 16 + iota], mse_v)
                    hi = pid_v >= HP
                    col = pid_v - jnp.where(hi, HP, 0)
                    val = jnp.where(hi, 65536, 1)
                    plsc.addupdate_scatter(cnt, [col * 16 + iota], val)
                return carry
            lax.fori_loop(0, GROUPS // 10, gb, 0)

        issue(wid, m0, p0)

        def pair(j, carry):
            c0 = wid + (2 * j) * NW
            c1 = c0 + NW
            c2 = c1 + NW
            @pl.when(c0 < nch)
            def _():
                drain(c0, m0, p0)
                @pl.when(c1 < nch)
                def _():
                    issue(c1, m1, p1)
                process(m0, p0)
                @pl.when(c1 < nch)
                def _():
                    drain(c1, m1, p1)
                    @pl.when(c2 < nch)
                    def _():
                        issue(c2, m0, p0)
                    process(m1, p1)
            return carry

        lax.fori_loop(0, (nch + 2 * NW - 1) // (2 * NW), pair, 0)

        def red_acc(b, carry):
            base = b * 256
            v = plsc.load_gather(acc, [iota16 + base])
            for r in range(1, 16):
                v = v + plsc.load_gather(acc, [iota16 + (base + r)])
            acc[pl.ds(b * 16, 16)] = v
            return carry

        def red_cnt(b, carry):
            base = b * 256
            v = plsc.load_gather(cnt, [iota16 + base])
            for r in range(1, 16):
                v = v + plsc.load_gather(cnt, [iota16 + (base + r)])
            cnt[pl.ds(b * 16, 16)] = v
            return carry

        lax.fori_loop(0, P // 16, red_acc, 0)
        lax.fori_loop(0, HP // 16, red_cnt, 0)
        pltpu.sync_copy(acc.at[pl.ds(0, P)], mse_out.at[wid])
        pltpu.sync_copy(cnt.at[pl.ds(0, HP)], cnt_out.at[wid])

    return _sc_body


_sc_segment = functools.partial(
    pl.kernel,
    out_type=(jax.ShapeDtypeStruct((NW, P), jnp.float32),
              jax.ShapeDtypeStruct((NW, HP), jnp.int32)),
    mesh=plsc.VectorSubcoreMesh(core_axis_name="c", subcore_axis_name="s"),
    scratch_types=[
        pltpu.VMEM((16 * P,), jnp.float32),
        pltpu.VMEM((16 * HP,), jnp.int32),
        pltpu.VMEM((CH,), jnp.float32),
        pltpu.VMEM((CH,), jnp.float32),
        pltpu.VMEM((CH,), jnp.int32),
        pltpu.VMEM((CH,), jnp.int32),
        pltpu.SemaphoreType.DMA,
    ],
    compiler_params=pltpu.CompilerParams(needs_layout_passes=False,
                                         use_tc_tiling_on_sc=False),
)(_make_sc_body(NS // CH))


def _final_body(mse_a, mse_b, cnt_a, cnt_b, out_ref):
    sum_mse = (jnp.sum(mse_a[...], axis=0, keepdims=True)
               + jnp.sum(mse_b[...], axis=0, keepdims=True))
    low = jnp.zeros((1, HP), jnp.float32)
    high = jnp.zeros((1, HP), jnp.float32)
    for ref in (cnt_a, cnt_b):
        packed = ref[...]
        low = low + jnp.sum((packed & 0xFFFF).astype(jnp.float32),
                            axis=0, keepdims=True)
        high = high + jnp.sum(
            (lax.shift_right_logical(packed, 16) & 0xFFFF)
            .astype(jnp.float32), axis=0, keepdims=True)
    counts = jnp.concatenate([low, high], axis=1)
    pids = lax.broadcasted_iota(jnp.int32, (1, P), 1).astype(jnp.float32)
    present = (counts > 0.0) & (pids != 0.0)
    xi_sum = pids * counts
    weighted = pids * sum_mse
    terms = jnp.where(present,
                      weighted / jnp.where(xi_sum > 0.0, xi_sum, 1.0),
                      0.0)
    k_cnt = jnp.sum(present.astype(jnp.float32))
    out_ref[0, 0] = 100.0 * jnp.sum(terms) / k_cnt


def kernel(W, beta, H, pred, Y, particle_id, track_params, reconstructable):
    parts = []
    eps = jnp.float32(0.0)
    zero_i = jnp.int32(0)
    for s in range(NSLICE):
        lo, hi = s * NS, (s + 1) * NS
        mse = jnp.sum((pred[lo:hi] - track_params[lo:hi] + eps) ** 2, axis=1)
        pid_eff = jnp.where(reconstructable[lo:hi] > zero_i,
                            particle_id[lo:hi], 0)
        eps = lax.optimization_barrier(jnp.minimum(jnp.min(mse), 0.0))
        zero_i = lax.optimization_barrier(jnp.minimum(jnp.min(pid_eff), 0))
        parts.append(_sc_segment(mse, pid_eff))
    (mse_a, cnt_a), (mse_b, cnt_b) = parts
    out = pl.pallas_call(
        _final_body,
        out_shape=jax.ShapeDtypeStruct((1, 1), jnp.float32),
        out_specs=pl.BlockSpec(memory_space=pltpu.SMEM),
    )(mse_a, mse_b, cnt_a, cnt_b)
    return out[0, 0]
